# baseline (speedup 1.0000x reference)
import functools

import jax
import jax.numpy as jnp
from jax import lax
from jax.experimental import pallas as pl
from jax.experimental.pallas import tpu as pltpu
from jax.experimental.pallas import tpu_sc as plsc

_B = 64
_NV = 4096
_L = 16
_W = 64
_NWORK = 8
_RPW = _B // _NWORK
_NWIN = _NV // _W


def _first_byte_in_window(v8, lane):
    w = plsc.bitcast(v8, jnp.int32)
    b0 = (w & 0xFF) != 0
    b1 = (w & 0xFF00) != 0
    b2 = (w & 0xFF0000) != 0
    byte_off = jnp.where(b0, 0, jnp.where(b1, 1, jnp.where(b2, 2, 3)))
    idx = jnp.where(w != 0, 4 * lane + byte_off, _NV)
    return jnp.min(idx)


def _make_sc_kernel():
    mesh = plsc.VectorSubcoreMesh(
        core_axis_name="c", subcore_axis_name="s", num_cores=1)

    @functools.partial(
        pl.kernel,
        mesh=mesh,
        out_type=jax.ShapeDtypeStruct((_B,), jnp.int32),
        scratch_types=[
            pltpu.VMEM((_RPW, _W), jnp.int8),
            pltpu.VMEM((_NV,), jnp.int8),
            pltpu.VMEM((_L,), jnp.int32),
            pltpu.SemaphoreType.DMA,
        ],
        compiler_params=pltpu.CompilerParams(
            needs_layout_passes=False, use_tc_tiling_on_sc=False,
            skip_device_barrier=True),
    )
    def sc_first_valid(mask_hbm, out_hbm, head_v, row_v, res_v, sem):
        sid = lax.axis_index("s")

        @pl.when(sid < _NWORK)
        def _():
            base = pl.multiple_of(sid * _RPW, _RPW)
            pltpu.async_copy(
                mask_hbm.at[pl.ds(base, _RPW), pl.ds(0, _W)], head_v,
                sem).wait()

            lane = lax.broadcasted_iota(jnp.int32, (_L,), 0)
            acc = jnp.zeros((_L,), jnp.int32)
            for r in range(_RPW):
                found = _first_byte_in_window(head_v[r], lane)

                def _fallback(_, r=r):
                    pltpu.sync_copy(mask_hbm.at[base + r], row_v)

                    def cond(st):
                        j, f = st
                        return jnp.logical_and(f >= _NV, j < _NWIN)

                    def body(st):
                        j, f = st
                        hit = _first_byte_in_window(
                            row_v[pl.ds(j * _W, _W)], lane)
                        f = jnp.where(hit < _NV, j * _W + hit, f)
                        return j + 1, f

                    _, f = lax.while_loop(
                        cond, body, (jnp.int32(1), jnp.int32(_NV)))
                    return f

                found = lax.cond(found >= _NV, _fallback,
                                 lambda _, found=found: found, 0)
                acc = jnp.where(lane == r, found, acc)

            res_v[...] = acc
            pltpu.sync_copy(res_v.at[pl.ds(0, _RPW)],
                            out_hbm.at[pl.ds(base, _RPW)])

    return sc_first_valid


_sc_first_valid = _make_sc_kernel()


def kernel(states, mask):
    del states
    return _sc_first_valid(mask.view(jnp.int8))


# speedup vs baseline: 1.0185x; 1.0185x over previous
import functools

import jax
import jax.numpy as jnp
from jax import lax
from jax.experimental import pallas as pl
from jax.experimental.pallas import tpu as pltpu
from jax.experimental.pallas import tpu_sc as plsc

_B = 64
_NV = 4096
_L = 16
_NWORK = 8
_RPW = _B // _NWORK
_NCHUNK = _NV // _L


def _make_sc_kernel():
    mesh = plsc.VectorSubcoreMesh(
        core_axis_name="c", subcore_axis_name="s", num_cores=1)

    @functools.partial(
        pl.kernel,
        mesh=mesh,
        out_type=jax.ShapeDtypeStruct((_B,), jnp.int32),
        scratch_types=[
            pltpu.VMEM((_RPW, _L), jnp.int32),
            pltpu.VMEM((_NV,), jnp.int32),
            pltpu.VMEM((_L,), jnp.int32),
            pltpu.SemaphoreType.DMA,
        ],
        compiler_params=pltpu.CompilerParams(needs_layout_passes=False),
    )
    def sc_first_valid(mask_hbm, out_hbm, head_v, row_v, res_v, sem):
        sid = lax.axis_index("s")

        @pl.when(sid < _NWORK)
        def _():
            base = pl.multiple_of(sid * _RPW, _RPW)
            copies = [
                pltpu.async_copy(
                    mask_hbm.at[base + r, pl.ds(0, _L)], head_v.at[r], sem)
                for r in range(_RPW)
            ]
            for cp in copies:
                cp.wait()

            lane = lax.broadcasted_iota(jnp.int32, (_L,), 0)
            acc = jnp.zeros((_L,), jnp.int32)
            for r in range(_RPW):
                head = head_v[r]
                found = plsc.all_reduce_ffs(head != 0)[0]

                def _fallback(_, r=r):
                    pltpu.sync_copy(mask_hbm.at[base + r], row_v)

                    def cond(st):
                        j, f = st
                        return jnp.logical_and(f >= _NV, j < _NCHUNK)

                    def body(st):
                        j, f = st
                        vv = row_v[pl.ds(j * _L, _L)]
                        hit = plsc.all_reduce_ffs(vv != 0)[0]
                        f = jnp.where(hit < _L, j * _L + hit, f)
                        return j + 1, f

                    _, f = lax.while_loop(
                        cond, body, (jnp.int32(1), jnp.int32(_NV)))
                    return f

                found = lax.cond(found >= _L, _fallback,
                                 lambda _, found=found: found, 0)
                acc = jnp.where(lane == r, found, acc)

            res_v[...] = acc
            pltpu.sync_copy(res_v.at[pl.ds(0, _RPW)],
                            out_hbm.at[pl.ds(base, _RPW)])

    return sc_first_valid


_sc_first_valid = _make_sc_kernel()


def kernel(states, mask):
    del states
    return _sc_first_valid(mask.astype(jnp.int32))


# speedup vs baseline: 1.0517x; 1.0326x over previous
import functools

import jax
import jax.numpy as jnp
from jax import lax
from jax.experimental import pallas as pl
from jax.experimental.pallas import tpu as pltpu
from jax.experimental.pallas import tpu_sc as plsc

_B = 64
_NV = 4096
_L = 16
_NWORK = 8
_RPW = _B // _NWORK
_NCHUNK = _NV // _L


def _make_sc_kernel():
    mesh = plsc.VectorSubcoreMesh(
        core_axis_name="c", subcore_axis_name="s", num_cores=1)

    @functools.partial(
        pl.kernel,
        mesh=mesh,
        out_type=jax.ShapeDtypeStruct((_B,), jnp.int32),
        scratch_types=[
            pltpu.VMEM((_RPW, _L), jnp.int32),
            pltpu.VMEM((_NV,), jnp.int32),
            pltpu.VMEM((_L,), jnp.int32),
            pltpu.SemaphoreType.DMA,
        ],
        compiler_params=pltpu.CompilerParams(needs_layout_passes=False),
    )
    def sc_first_valid(mask_hbm, out_hbm, head_v, row_v, res_v, sem):
        sid = lax.axis_index("s")

        @pl.when(sid < _NWORK)
        def _():
            base = pl.multiple_of(sid * _RPW, _RPW)
            copies = [
                pltpu.async_copy(
                    mask_hbm.at[base + r, pl.ds(0, _L)], head_v.at[r], sem)
                for r in range(_RPW)
            ]
            for cp in copies:
                cp.wait()

            lane = lax.broadcasted_iota(jnp.int32, (_L,), 0)
            acc = jnp.zeros((_L,), jnp.int32)
            for r in range(_RPW):
                found = plsc.all_reduce_ffs(head_v[r] != 0)[0]
                acc = jnp.where(lane == r,
                                jnp.where(found < _L, found, -1), acc)

            def any_missing(a):
                return plsc.all_reduce_ffs(a < 0)[0] < _L

            def resolve_one(a):
                r = plsc.all_reduce_ffs(a < 0)[0]
                pltpu.sync_copy(mask_hbm.at[base + r], row_v)

                def cond(st):
                    j, f = st
                    return jnp.logical_and(f >= _NV, j < _NCHUNK)

                def body(st):
                    j, f = st
                    hit = plsc.all_reduce_ffs(
                        row_v[pl.ds(j * _L, _L)] != 0)[0]
                    f = jnp.where(hit < _L, j * _L + hit, f)
                    return j + 1, f

                _, f = lax.while_loop(
                    cond, body, (jnp.int32(0), jnp.int32(_NV)))
                return jnp.where(lane == r, f, a)

            acc = lax.while_loop(any_missing, resolve_one, acc)

            res_v[...] = acc
            pltpu.sync_copy(res_v.at[pl.ds(0, _RPW)],
                            out_hbm.at[pl.ds(base, _RPW)])

    return sc_first_valid


_sc_first_valid = _make_sc_kernel()


def kernel(states, mask):
    del states
    return _sc_first_valid(mask.astype(jnp.int32))
